# baseline (speedup 1.0000x reference)
import jax
import jax.numpy as jnp
from jax.experimental import pallas as pl
from jax.experimental.pallas import tpu as pltpu

_B, _C, _H, _W = 64, 768, 16, 16
_HW = _H * _W
_E, _TOPK = 8, 2
_BB = 4
_NCHUNK = _B // _BB
_NBUF = 8


def _router_kernel(x_hbm, w_ref, b_ref, idx_ref, wgt_ref, buf, st_sc, sem):
    def start(chunk, slot):
        pltpu.make_async_copy(
            x_hbm.at[pl.ds(chunk * _BB, _BB)], buf.at[slot], sem.at[slot]
        ).start()

    for k in range(_NBUF):
        start(k, k)

    for chunk in range(_NCHUNK):
        slot = chunk % _NBUF
        pltpu.make_async_copy(
            x_hbm.at[pl.ds(chunk * _BB, _BB)], buf.at[slot], sem.at[slot]
        ).wait()
        xb = buf[slot]
        pooled = jnp.mean(xb, axis=1)
        st_sc[:, pl.ds(chunk * _BB, _BB)] = jax.lax.dot_general(
            w_ref[...], pooled,
            dimension_numbers=(((1,), (1,)), ((), ())),
            preferred_element_type=jnp.float32)
        nxt = chunk + _NBUF
        if nxt < _NCHUNK:
            start(nxt, slot)

    b_col = jax.lax.dot_general(
        jnp.eye(_E, dtype=jnp.float32), b_ref[...],
        dimension_numbers=(((1,), (1,)), ((), ())),
        preferred_element_type=jnp.float32)
    st = st_sc[...] + b_col

    m = jnp.max(st, axis=0, keepdims=True)
    ex = jnp.exp(st - m)
    probs = ex / jnp.sum(ex, axis=0, keepdims=True)

    rows = jax.lax.broadcasted_iota(jnp.int32, (_E, _B), 0)
    p1 = jnp.max(probs, axis=0, keepdims=True)
    i1 = jnp.min(jnp.where(probs == p1, rows, _E), axis=0, keepdims=True)
    masked = jnp.where(rows == i1, -jnp.inf, probs)
    p2 = jnp.max(masked, axis=0, keepdims=True)
    i2 = jnp.min(jnp.where(masked == p2, rows, _E), axis=0, keepdims=True)
    s = p1 + p2

    wgt_ref[0:1, :] = p1 / s
    wgt_ref[1:2, :] = p2 / s
    idx_ref[0:1, :] = i1
    idx_ref[1:2, :] = i2


def kernel(x, fc_w, fc_b):
    xr = jnp.transpose(x, (0, 2, 3, 1)).reshape(_B, _HW, _C)
    br = fc_b.reshape(1, _E)
    idx_t, wgt_t = pl.pallas_call(
        _router_kernel,
        in_specs=[
            pl.BlockSpec(memory_space=pltpu.MemorySpace.HBM),
            pl.BlockSpec((_E, _C), lambda: (0, 0)),
            pl.BlockSpec((1, _E), lambda: (0, 0)),
        ],
        out_specs=[
            pl.BlockSpec((_TOPK, _B), lambda: (0, 0)),
            pl.BlockSpec((_TOPK, _B), lambda: (0, 0)),
        ],
        out_shape=[
            jax.ShapeDtypeStruct((_TOPK, _B), jnp.int32),
            jax.ShapeDtypeStruct((_TOPK, _B), jnp.float32),
        ],
        scratch_shapes=[
            pltpu.VMEM((_NBUF, _BB, _HW, _C), jnp.float32),
            pltpu.VMEM((_E, _B), jnp.float32),
            pltpu.SemaphoreType.DMA((_NBUF,)),
        ],
    )(xr, fc_w, br)
    return idx_t.T, wgt_t.T


# speedup vs baseline: 1.0097x; 1.0097x over previous
import jax
import jax.numpy as jnp
from jax.experimental import pallas as pl
from jax.experimental.pallas import tpu as pltpu

_B, _C, _H, _W = 64, 768, 16, 16
_HW = _H * _W
_E, _TOPK = 8, 2
_BB = 4
_NCHUNK = _B // _BB
_NBUF = 8


def _router_kernel(x_hbm, w_ref, b_ref, idx_ref, wgt_ref, buf, pooled_sc, sem):
    def start(chunk, slot):
        pltpu.make_async_copy(
            x_hbm.at[pl.ds(chunk * _BB, _BB)], buf.at[slot], sem.at[slot]
        ).start()

    for k in range(_NBUF):
        start(k, k)

    for chunk in range(_NCHUNK):
        slot = chunk % _NBUF
        pltpu.make_async_copy(
            x_hbm.at[pl.ds(chunk * _BB, _BB)], buf.at[slot], sem.at[slot]
        ).wait()
        xb = buf[slot]
        pooled_sc[pl.ds(chunk * _BB, _BB), :] = jnp.mean(xb, axis=1)
        nxt = chunk + _NBUF
        if nxt < _NCHUNK:
            start(nxt, slot)

    st = jax.lax.dot_general(
        w_ref[...], pooled_sc[...],
        dimension_numbers=(((1,), (1,)), ((), ())),
        preferred_element_type=jnp.float32)
    b_col = jax.lax.dot_general(
        jnp.eye(_E, dtype=jnp.float32), b_ref[...],
        dimension_numbers=(((1,), (1,)), ((), ())),
        preferred_element_type=jnp.float32)
    st = st + b_col

    m = jnp.max(st, axis=0, keepdims=True)
    ex = jnp.exp(st - m)
    probs = ex / jnp.sum(ex, axis=0, keepdims=True)

    rows = jax.lax.broadcasted_iota(jnp.int32, (_E, _B), 0)
    p1 = jnp.max(probs, axis=0, keepdims=True)
    i1 = jnp.min(jnp.where(probs == p1, rows, _E), axis=0, keepdims=True)
    masked = jnp.where(rows == i1, -jnp.inf, probs)
    p2 = jnp.max(masked, axis=0, keepdims=True)
    i2 = jnp.min(jnp.where(masked == p2, rows, _E), axis=0, keepdims=True)
    s = p1 + p2

    wgt_ref[0:1, :] = p1 / s
    wgt_ref[1:2, :] = p2 / s
    idx_ref[0:1, :] = i1
    idx_ref[1:2, :] = i2


def kernel(x, fc_w, fc_b):
    xr = jnp.transpose(x, (0, 2, 3, 1)).reshape(_B, _HW, _C)
    br = fc_b.reshape(1, _E)
    idx_t, wgt_t = pl.pallas_call(
        _router_kernel,
        in_specs=[
            pl.BlockSpec(memory_space=pltpu.MemorySpace.HBM),
            pl.BlockSpec((_E, _C), lambda: (0, 0)),
            pl.BlockSpec((1, _E), lambda: (0, 0)),
        ],
        out_specs=[
            pl.BlockSpec((_TOPK, _B), lambda: (0, 0)),
            pl.BlockSpec((_TOPK, _B), lambda: (0, 0)),
        ],
        out_shape=[
            jax.ShapeDtypeStruct((_TOPK, _B), jnp.int32),
            jax.ShapeDtypeStruct((_TOPK, _B), jnp.float32),
        ],
        scratch_shapes=[
            pltpu.VMEM((_NBUF, _BB, _HW, _C), jnp.float32),
            pltpu.VMEM((_B, _C), jnp.float32),
            pltpu.SemaphoreType.DMA((_NBUF,)),
        ],
    )(xr, fc_w, br)
    return idx_t.T, wgt_t.T
